# Initial kernel scaffold; baseline (speedup 1.0000x reference)
import jax
import jax.numpy as jnp
from jax.experimental import pallas as pl


def kernel(w1, b1, w2, b2, w3, b3, w4, b4, w5, b5, x):
    raise NotImplementedError("write your pallas kernel here")


# speedup vs baseline: 17.2181x; 17.2181x over previous
import functools

import jax
import jax.numpy as jnp
from jax.experimental import pallas as pl
from jax.experimental.pallas import tpu as pltpu

_NEG = 0.2
_EPS = 1e-5
_BF = jnp.bfloat16
_F32 = jnp.float32


def _lrelu(y):
    return jnp.where(y >= 0, y, _NEG * y)


def _dot3(lhs, w):
    return jax.lax.dot_general(lhs, w, (((2,), (0,)), ((), ())),
                               preferred_element_type=_F32)


def _store_padded(o_ref, y, ho, wo, c):
    hp, wp = ho + 2, wo + 2
    o_ref[0, 0:1, :, :] = jnp.zeros((1, wp, c), _BF)
    o_ref[0, hp - 1:hp, :, :] = jnp.zeros((1, wp, c), _BF)
    o_ref[0, 1:ho + 1, 0:1, :] = jnp.zeros((ho, 1, c), _BF)
    o_ref[0, 1:ho + 1, wp - 1:wp, :] = jnp.zeros((ho, 1, c), _BF)
    o_ref[0, 1:ho + 1, 1:wo + 1, :] = y


def _conv1_kernel(x_ref, w_ref, b_ref, o_ref):
    xv = x_ref[0]
    y = None
    for t in range(4):
        a, b = t // 2, t % 2
        d = _dot3(xv[a:a + 128, b:b + 128, :], w_ref[t])
        y = d if y is None else y + d
    y = _lrelu(y + b_ref[...])
    _store_padded(o_ref, y.astype(_BF), 128, 128, 64)


def _conv_s2_in_kernel(x_ref, w_ref, b_ref, o_ref, *, ho):
    xe = x_ref[0, :, 0]
    xo = x_ref[0, :, 1]
    ae = jnp.concatenate([xe[:, 0:ho, :], xe[:, 1:ho + 1, :]], axis=-1)
    ao = jnp.concatenate([xo[:, 0:ho, :], xo[:, 1:ho + 1, :]], axis=-1)
    y = _dot3(ae[0:ho], w_ref[0])
    y = y + _dot3(ao[0:ho], w_ref[1])
    y = y + _dot3(ae[1:ho + 1], w_ref[2])
    y = y + _dot3(ao[1:ho + 1], w_ref[3])
    y = (y + b_ref[...]).astype(_BF).astype(_F32)
    mean = jnp.mean(y, axis=(0, 1), keepdims=True)
    xc = y - mean
    var = jnp.mean(xc * xc, axis=(0, 1), keepdims=True)
    yn = _lrelu(xc * jax.lax.rsqrt(var + _EPS)).astype(_BF)
    _store_padded(o_ref, yn, ho, ho, yn.shape[-1])


def _conv4_head_kernel(x_ref, w4_ref, b4_ref, w5_ref, o_ref):
    xv = x_ref[0]
    a = jnp.concatenate(
        [xv[:, 0:31, :], xv[:, 1:32, :], xv[:, 2:33, :], xv[:, 3:34, :]],
        axis=-1)
    y = None
    for i in range(4):
        d = _dot3(a[i:i + 31], w4_ref[i])
        y = d if y is None else y + d
    y = (y + b4_ref[...]).astype(_BF).astype(_F32)
    mean = jnp.mean(y, axis=(0, 1), keepdims=True)
    xc = y - mean
    var = jnp.mean(xc * xc, axis=(0, 1), keepdims=True)
    y4 = _lrelu(xc * jax.lax.rsqrt(var + _EPS)).astype(_BF)
    parts = []
    for i in range(4):
        r0, r1 = max(0, i - 1), min(31, i + 29)
        for j in range(4):
            c0, c1 = max(0, j - 1), min(31, j + 29)
            s = jnp.sum(y4[r0:r1, c0:c1, :].astype(_F32), axis=(0, 1),
                        keepdims=True)
            parts.append(s[0])
    sm = jnp.concatenate(parts, axis=0)
    pooled = (sm * (1.0 / 900.0)).astype(_BF).astype(_F32)
    prod = pooled * w5_ref[...].astype(_F32)
    v = jnp.sum(prod, axis=1, keepdims=True)
    val = jnp.sum(v, axis=0, keepdims=True)
    o_ref[...] = jnp.broadcast_to(val, (1, 1, 128))


def _wmat(w):
    kh, kw = w.shape[2], w.shape[3]
    return jnp.transpose(w, (2, 3, 1, 0)).reshape(
        kh, kw * w.shape[1], w.shape[0]).astype(_BF)


def kernel(w1, b1, w2, b2, w3, b3, w4, b4, w5, b5, x):
    B = x.shape[0]
    par = pltpu.CompilerParams(dimension_semantics=("parallel",))

    xh = jnp.transpose(x.astype(_BF), (0, 2, 3, 1))
    xp = jnp.pad(xh, ((0, 0), (1, 1), (1, 1), (0, 0)))
    x1 = xp.reshape(B, 129, 2, 129, 2, 3).transpose(0, 1, 3, 2, 4, 5)
    x1 = x1.reshape(B, 129, 129, 12)
    wt1 = jnp.transpose(w1, (2, 3, 1, 0))
    w1p = wt1.reshape(2, 2, 2, 2, 3, 64).transpose(0, 2, 1, 3, 4, 5)
    w1p = w1p.reshape(4, 12, 64).astype(_BF)
    out1 = pl.pallas_call(
        _conv1_kernel,
        out_shape=jax.ShapeDtypeStruct((B, 130, 130, 64), _BF),
        grid=(B,),
        in_specs=[
            pl.BlockSpec((1, 129, 129, 12), lambda b: (b, 0, 0, 0)),
            pl.BlockSpec((4, 12, 64), lambda b: (0, 0, 0)),
            pl.BlockSpec((1, 1, 64), lambda b: (0, 0, 0)),
        ],
        out_specs=pl.BlockSpec((1, 130, 130, 64), lambda b: (b, 0, 0, 0)),
        compiler_params=par,
    )(x1, w1p, b1.reshape(1, 1, 64).astype(_F32))

    x2 = out1.reshape(B, 65, 2, 65, 128)
    out2 = pl.pallas_call(
        functools.partial(_conv_s2_in_kernel, ho=64),
        out_shape=jax.ShapeDtypeStruct((B, 66, 66, 128), _BF),
        grid=(B,),
        in_specs=[
            pl.BlockSpec((1, 65, 2, 65, 128), lambda b: (b, 0, 0, 0, 0)),
            pl.BlockSpec((4, 256, 128), lambda b: (0, 0, 0)),
            pl.BlockSpec((1, 1, 128), lambda b: (0, 0, 0)),
        ],
        out_specs=pl.BlockSpec((1, 66, 66, 128), lambda b: (b, 0, 0, 0)),
        compiler_params=par,
    )(x2, _wmat(w2), b2.reshape(1, 1, 128).astype(_F32))

    x3 = out2.reshape(B, 33, 2, 33, 256)
    out3 = pl.pallas_call(
        functools.partial(_conv_s2_in_kernel, ho=32),
        out_shape=jax.ShapeDtypeStruct((B, 34, 34, 256), _BF),
        grid=(B,),
        in_specs=[
            pl.BlockSpec((1, 33, 2, 33, 256), lambda b: (b, 0, 0, 0, 0)),
            pl.BlockSpec((4, 512, 256), lambda b: (0, 0, 0)),
            pl.BlockSpec((1, 1, 256), lambda b: (0, 0, 0)),
        ],
        out_specs=pl.BlockSpec((1, 34, 34, 256), lambda b: (b, 0, 0, 0)),
        compiler_params=par,
    )(x3, _wmat(w3), b3.reshape(1, 1, 256).astype(_F32))

    w5p = jnp.transpose(w5, (2, 3, 1, 0)).reshape(16, 512).astype(_BF)
    out45 = pl.pallas_call(
        _conv4_head_kernel,
        out_shape=jax.ShapeDtypeStruct((B, 1, 128), _F32),
        grid=(B,),
        in_specs=[
            pl.BlockSpec((1, 34, 34, 256), lambda b: (b, 0, 0, 0)),
            pl.BlockSpec((4, 1024, 512), lambda b: (0, 0, 0)),
            pl.BlockSpec((1, 1, 512), lambda b: (0, 0, 0)),
            pl.BlockSpec((16, 512), lambda b: (0, 0)),
        ],
        out_specs=pl.BlockSpec((1, 1, 128), lambda b: (b, 0, 0)),
        compiler_params=par,
    )(out3, _wmat(w4), b4.reshape(1, 1, 512).astype(_F32), w5p)

    return out45[:, 0, :1] + b5.reshape(1, 1).astype(_F32)
